# Initial kernel scaffold; baseline (speedup 1.0000x reference)
import jax
import jax.numpy as jnp
from jax.experimental import pallas as pl


def kernel(x, edge_index, edge_attr, layer_params, readout_params):
    raise NotImplementedError("write your pallas kernel here")


# speedup vs baseline: 1.2786x; 1.2786x over previous
import functools

import jax
import jax.numpy as jnp
from jax.experimental import pallas as pl

N_NODES = 10000
E_EDGES = 160000
D_NODE = 256
D_EDGE = 16
HID = 256
SCALE = 1.0 / (HID ** 0.5)

_ROWS = 1000


def _proj_body(h_ref, w_ref, b_ref, o_ref):
    o_ref[...] = jnp.dot(h_ref[...], w_ref[...],
                         preferred_element_type=jnp.float32) + b_ref[...]


@functools.partial(jax.jit, static_argnames=())
def _proj(h, w_cat, b_cat):
    n, d = h.shape
    k = w_cat.shape[1]
    return pl.pallas_call(
        _proj_body,
        grid=(n // _ROWS,),
        in_specs=[
            pl.BlockSpec((_ROWS, d), lambda i: (i, 0)),
            pl.BlockSpec((d, k), lambda i: (0, 0)),
            pl.BlockSpec((1, k), lambda i: (0, 0)),
        ],
        out_specs=pl.BlockSpec((_ROWS, k), lambda i: (i, 0)),
        out_shape=jax.ShapeDtypeStruct((n, k), jnp.float32),
    )(h, w_cat, b_cat)


def _layer(h, src, dst, edge_attr, p):
    w_cat = jnp.concatenate(
        [p["Wq"].T, p["Wk"].T, p["Wv"].T, p["Ws"].T], axis=1)
    b_cat = jnp.concatenate(
        [p["bq"], p["bk"], p["bv"], p["bs"]])[None, :]
    qkvs = _proj(h, w_cat, b_cat)
    q = qkvs[:, 0:HID]
    k = qkvs[:, HID:2 * HID]
    v = qkvs[:, 2 * HID:3 * HID]
    skip = qkvs[:, 3 * HID:4 * HID]
    qw = q @ p["We"]

    alpha = (jnp.sum(q[dst] * k[src], axis=-1)
             + jnp.sum(qw[dst] * edge_attr, axis=-1)) * SCALE
    s = jnp.exp(alpha)
    denom = jax.ops.segment_sum(s, dst, num_segments=N_NODES)
    w_e = s / (denom[dst] + 1e-16)
    agg = jax.ops.segment_sum(v[src] * w_e[:, None], dst,
                              num_segments=N_NODES)
    agg16 = jax.ops.segment_sum(edge_attr * w_e[:, None], dst,
                                num_segments=N_NODES)
    return jax.nn.relu(agg + agg16 @ p["We"].T + skip)


def kernel(x, edge_index, edge_attr, layer_params, readout_params):
    src = edge_index[0]
    dst = edge_index[1]
    h = x
    for p in layer_params:
        h = _layer(h, src, dst, edge_attr, p)
    pooled = jnp.mean(h, axis=0, keepdims=True)
    r = jax.nn.relu(pooled @ readout_params["W1"].T + readout_params["b1"])
    out = r @ readout_params["W2"].T + readout_params["b2"]
    return out.squeeze(-1)


# speedup vs baseline: 1.4319x; 1.1199x over previous
import jax
import jax.numpy as jnp
from jax import lax
from jax.experimental import pallas as pl
from jax.experimental.pallas import tpu as pltpu
from jax.experimental.pallas import tpu_sc as plsc

N = 10000
NP = 10240
E = 160000
EP = 163840
D = 256
DE = 16
HID = 256
HH = 128
SCALE = 1.0 / (HID ** 0.5)

NTILES = 32
TE = EP // NTILES
CH = 64
NCH = TE // CH
RPT = NP // 16
TCB = NP // 8

_MESH = plsc.VectorSubcoreMesh(core_axis_name="c", subcore_axis_name="s")
_SC_PARAMS = pltpu.CompilerParams(use_tc_tiling_on_sc=False,
                                  needs_layout_passes=False)


def _proj_body(h_ref, w_ref, b_ref, we_ref,
               q_ref, k_ref, vlo_ref, vhi_ref, skip_ref, qw_ref):
    big = jnp.dot(h_ref[...], w_ref[...],
                  preferred_element_type=jnp.float32) + b_ref[...]
    q = big[:, 0:HID]
    q_ref[...] = q
    k_ref[...] = big[:, HID:2 * HID]
    vlo_ref[...] = big[:, 2 * HID:2 * HID + HH]
    vhi_ref[...] = big[:, 2 * HID + HH:3 * HID]
    skip_ref[...] = big[:, 3 * HID:4 * HID]
    qw_ref[...] = jnp.dot(q, we_ref[...], preferred_element_type=jnp.float32)


def _proj(h, wcat, bcat, we):
    outs = [
        jax.ShapeDtypeStruct((NP, HID), jnp.float32),
        jax.ShapeDtypeStruct((NP, HID), jnp.float32),
        jax.ShapeDtypeStruct((NP, HH), jnp.float32),
        jax.ShapeDtypeStruct((NP, HH), jnp.float32),
        jax.ShapeDtypeStruct((NP, HID), jnp.float32),
        jax.ShapeDtypeStruct((NP, DE), jnp.float32),
    ]
    return pl.pallas_call(
        _proj_body,
        grid=(NP // TCB,),
        in_specs=[
            pl.BlockSpec((TCB, D), lambda i: (i, 0)),
            pl.BlockSpec((D, 4 * HID), lambda i: (0, 0)),
            pl.BlockSpec((1, 4 * HID), lambda i: (0, 0)),
            pl.BlockSpec((D, DE), lambda i: (0, 0)),
        ],
        out_specs=[
            pl.BlockSpec((TCB, HID), lambda i: (i, 0)),
            pl.BlockSpec((TCB, HID), lambda i: (i, 0)),
            pl.BlockSpec((TCB, HH), lambda i: (i, 0)),
            pl.BlockSpec((TCB, HH), lambda i: (i, 0)),
            pl.BlockSpec((TCB, HID), lambda i: (i, 0)),
            pl.BlockSpec((TCB, DE), lambda i: (i, 0)),
        ],
        out_shape=outs,
    )(h, wcat, bcat, we)


def _pass1_body(q_hbm, k_hbm, qw_hbm, ea_hbm, src2_hbm, dst2_hbm,
                s_out, agg_out,
                idx_src, idx_dst, qbuf, kbuf, qwbuf, eabuf,
                stage32, sbuf, agg_sp):
    c = lax.axis_index("c")
    s_ax = lax.axis_index("s")
    wid = c * 16 + s_ax
    ebase = wid * TE
    rbase = s_ax * RPT

    zf = jnp.zeros((16,), jnp.float32)

    def _zstage(r, _):
        stage32[r, pl.ds(0, 16)] = zf
        stage32[r, pl.ds(16, 16)] = zf
        return 0
    lax.fori_loop(0, CH, _zstage, 0)

    for z in range(RPT // CH):
        pltpu.sync_copy(stage32, agg_sp.at[pl.ds(rbase + z * CH, CH)])
    plsc.subcore_barrier()

    lanes = lax.iota(jnp.int32, 16)

    def _chunk(ci, _):
        pltpu.sync_copy(src2_hbm.at[pl.ds(wid * NCH + ci, 1)], idx_src)
        pltpu.sync_copy(dst2_hbm.at[pl.ds(wid * NCH + ci, 1)], idx_dst)
        pltpu.sync_copy(q_hbm.at[idx_dst.at[0]], qbuf)
        pltpu.sync_copy(k_hbm.at[idx_src.at[0]], kbuf)
        pltpu.sync_copy(qw_hbm.at[idx_dst.at[0]], qwbuf)
        pltpu.sync_copy(ea_hbm.at[pl.ds(ebase + ci * CH, CH)], eabuf)

        for g in range(CH // 16):
            ids = g * 16 + lanes
            gid = ebase + ci * CH + ids
            mask = gid < E

            def _dot(f, acc):
                col = jnp.full((16,), f, jnp.int32)
                qv = plsc.load_gather(qbuf, [ids, col])
                kv = plsc.load_gather(kbuf, [ids, col])
                return acc + qv * kv
            acc = lax.fori_loop(0, HID, _dot, jnp.zeros((16,), jnp.float32),
                                unroll=8)
            for f in range(DE):
                col = jnp.full((16,), f, jnp.int32)
                acc = acc + (plsc.load_gather(qwbuf, [ids, col])
                             * plsc.load_gather(eabuf, [ids, col]))
            sv = jnp.where(mask, jnp.exp(acc * SCALE), 0.0)
            sbuf[pl.ds(g * 16, 16)] = sv

        def _egroup(g2, _):
            wv = sbuf[pl.ds(g2 * 16, 16)]
            for j16 in range(16):
                j = g2 * 16 + j16
                w = wv[j16]
                stage32[j, pl.ds(0, 16)] = eabuf[j, :] * w
                stage32[j, pl.ds(16, 16)] = jnp.full((16,), w, jnp.float32)
            return 0
        lax.fori_loop(0, CH // 16, _egroup, 0)

        pltpu.sync_copy(stage32, agg_sp.at[idx_dst.at[0]], add=True)
        pltpu.sync_copy(sbuf, s_out.at[pl.ds(ebase + ci * CH, CH)])
        return 0

    lax.fori_loop(0, NCH, _chunk, 0)

    plsc.subcore_barrier()
    pltpu.sync_copy(agg_sp.at[pl.ds(rbase, RPT)],
                    agg_out.at[c, pl.ds(rbase, RPT)])


def _pass1(q, k, qw, ea_p, src2, dst2):
    f = pl.kernel(
        _pass1_body,
        out_type=[
            jax.ShapeDtypeStruct((EP,), jnp.float32),
            jax.ShapeDtypeStruct((2, NP, 32), jnp.float32),
        ],
        mesh=_MESH,
        compiler_params=_SC_PARAMS,
        scratch_types=[
            pltpu.VMEM((1, CH), jnp.int32),
            pltpu.VMEM((1, CH), jnp.int32),
            pltpu.VMEM((CH, HID), jnp.float32),
            pltpu.VMEM((CH, HID), jnp.float32),
            pltpu.VMEM((CH, DE), jnp.float32),
            pltpu.VMEM((CH, DE), jnp.float32),
            pltpu.VMEM((CH, 32), jnp.float32),
            pltpu.VMEM((CH,), jnp.float32),
            pltpu.VMEM_SHARED((NP, 32), jnp.float32),
        ],
    )
    return f(q, k, qw, ea_p, src2, dst2)


def _pass2_body(vh_hbm, s_hbm, src2_hbm, dst2_hbm,
                acc_out,
                idx_src, idx_dst, vbuf, stage, sbuf, acc_sp):
    c = lax.axis_index("c")
    s_ax = lax.axis_index("s")
    wid = c * 16 + s_ax
    ebase = wid * TE
    rbase = s_ax * RPT

    zf = jnp.zeros((16,), jnp.float32)

    def _zstage(r, _):
        for b in range(HH // 16):
            stage[r, pl.ds(b * 16, 16)] = zf
        return 0
    lax.fori_loop(0, CH, _zstage, 0)
    for z in range(RPT // CH):
        pltpu.sync_copy(stage, acc_sp.at[pl.ds(rbase + z * CH, CH)])
    plsc.subcore_barrier()

    def _chunk(ci, _):
        pltpu.sync_copy(src2_hbm.at[pl.ds(wid * NCH + ci, 1)], idx_src)
        pltpu.sync_copy(dst2_hbm.at[pl.ds(wid * NCH + ci, 1)], idx_dst)
        pltpu.sync_copy(vh_hbm.at[idx_src.at[0]], vbuf)
        pltpu.sync_copy(s_hbm.at[pl.ds(ebase + ci * CH, CH)], sbuf)

        def _egroup(g2, _):
            wv = sbuf[pl.ds(g2 * 16, 16)]
            for j16 in range(16):
                j = g2 * 16 + j16
                w = wv[j16]
                for b in range(HH // 16):
                    stage[j, pl.ds(b * 16, 16)] = \
                        vbuf[j, pl.ds(b * 16, 16)] * w
            return 0
        lax.fori_loop(0, CH // 16, _egroup, 0)
        pltpu.sync_copy(stage, acc_sp.at[idx_dst.at[0]], add=True)
        return 0

    lax.fori_loop(0, NCH, _chunk, 0)
    plsc.subcore_barrier()
    pltpu.sync_copy(acc_sp.at[pl.ds(rbase, RPT)],
                    acc_out.at[c, pl.ds(rbase, RPT)])


def _pass2(vh, s, src2, dst2):
    f = pl.kernel(
        _pass2_body,
        out_type=jax.ShapeDtypeStruct((2, NP, HH), jnp.float32),
        mesh=_MESH,
        compiler_params=_SC_PARAMS,
        scratch_types=[
            pltpu.VMEM((1, CH), jnp.int32),
            pltpu.VMEM((1, CH), jnp.int32),
            pltpu.VMEM((CH, HH), jnp.float32),
            pltpu.VMEM((CH, HH), jnp.float32),
            pltpu.VMEM((CH,), jnp.float32),
            pltpu.VMEM_SHARED((NP, HH), jnp.float32),
        ],
    )
    return f(vh, s, src2, dst2)


def _combine_body(acclo_ref, acchi_ref, agg_ref, skip_ref,
                  wet_ref, out_ref):
    a = agg_ref[0] + agg_ref[1]
    den = a[:, 16:17]
    inv = 1.0 / (den + 1e-16)
    lo = (acclo_ref[0] + acclo_ref[1]) * inv
    hi = (acchi_ref[0] + acchi_ref[1]) * inv
    corr = jnp.dot(a[:, 0:DE], wet_ref[...],
                   preferred_element_type=jnp.float32)
    out_ref[...] = jax.nn.relu(
        jnp.concatenate([lo, hi], axis=1) + corr + skip_ref[...])


def _combine(acclo, acchi, agg, skip, wet):
    return pl.pallas_call(
        _combine_body,
        grid=(NP // TCB,),
        in_specs=[
            pl.BlockSpec((2, TCB, HH), lambda i: (0, i, 0)),
            pl.BlockSpec((2, TCB, HH), lambda i: (0, i, 0)),
            pl.BlockSpec((2, TCB, 32), lambda i: (0, i, 0)),
            pl.BlockSpec((TCB, HID), lambda i: (i, 0)),
            pl.BlockSpec((DE, HID), lambda i: (0, 0)),
        ],
        out_specs=pl.BlockSpec((TCB, HID), lambda i: (i, 0)),
        out_shape=jax.ShapeDtypeStruct((NP, HID), jnp.float32),
    )(acclo, acchi, agg, skip, wet)


def _readout_body(h_ref, w1_ref, b1_ref, w2_ref, b2_ref, out_ref):
    pooled = jnp.sum(h_ref[0:N, :], axis=0, keepdims=True) * (1.0 / N)
    r = jax.nn.relu(jnp.dot(pooled, w1_ref[...],
                            preferred_element_type=jnp.float32) + b1_ref[...])
    out_ref[...] = jnp.dot(r, w2_ref[...],
                           preferred_element_type=jnp.float32) + b2_ref[...]


def _readout(h, w1t, b1, w2t, b2):
    return pl.pallas_call(
        _readout_body,
        out_shape=jax.ShapeDtypeStruct((1, 1), jnp.float32),
    )(h, w1t, b1, w2t, b2)


def kernel(x, edge_index, edge_attr, layer_params, readout_params):
    src = edge_index[0]
    dst = edge_index[1]
    src2 = jnp.pad(src, (0, EP - E)).reshape(EP // CH, CH)
    dst2 = jnp.pad(dst, (0, EP - E)).reshape(EP // CH, CH)
    ea_p = jnp.pad(edge_attr, ((0, EP - E), (0, 0)))
    h = jnp.pad(x, ((0, NP - N), (0, 0)))

    for p in layer_params:
        wcat = jnp.concatenate(
            [p["Wq"].T, p["Wk"].T, p["Wv"].T, p["Ws"].T], axis=1)
        bcat = jnp.concatenate(
            [p["bq"], p["bk"], p["bv"], p["bs"]])[None, :]
        q, k, vlo, vhi, skip, qw = _proj(h, wcat, bcat, p["We"])
        s, agg = _pass1(q, k, qw, ea_p, src2, dst2)
        acclo = _pass2(vlo, s, src2, dst2)
        acchi = _pass2(vhi, s, src2, dst2)
        h = _combine(acclo, acchi, agg, skip, p["We"].T)

    rp = readout_params
    out = _readout(h, rp["W1"].T, rp["b1"][None, :], rp["W2"].T,
                   rp["b2"][None, :])
    return out[0]


# speedup vs baseline: 2.2418x; 1.5656x over previous
import jax
import jax.numpy as jnp
from jax import lax
from jax.experimental import pallas as pl
from jax.experimental.pallas import tpu as pltpu
from jax.experimental.pallas import tpu_sc as plsc

N = 10000
NP = 10240
E = 160000
EP = 163840
D = 256
DE = 16
HID = 256
HH = 128
SCALE = 1.0 / (HID ** 0.5)

NTILES = 32
TE = EP // NTILES
CH = 64
NCH = TE // CH
RPT = NP // 16
TCB = NP // 8

_MESH = plsc.VectorSubcoreMesh(core_axis_name="c", subcore_axis_name="s")
_SC_PARAMS = pltpu.CompilerParams(use_tc_tiling_on_sc=False,
                                  needs_layout_passes=False)


def _proj_body(h_ref, w_ref, b_ref, we_ref,
               q_ref, k_ref, vlo_ref, vhi_ref, skip_ref, qw_ref):
    big = jnp.dot(h_ref[...], w_ref[...],
                  preferred_element_type=jnp.float32) + b_ref[...]
    q = big[:, 0:HID]
    q_ref[...] = q
    k_ref[...] = big[:, HID:2 * HID]
    vlo_ref[...] = big[:, 2 * HID:2 * HID + HH]
    vhi_ref[...] = big[:, 2 * HID + HH:3 * HID]
    skip_ref[...] = big[:, 3 * HID:4 * HID]
    qw_ref[...] = jnp.dot(q, we_ref[...], preferred_element_type=jnp.float32)


def _proj(h, wcat, bcat, we):
    outs = [
        jax.ShapeDtypeStruct((NP, HID), jnp.float32),
        jax.ShapeDtypeStruct((NP, HID), jnp.float32),
        jax.ShapeDtypeStruct((NP, HH), jnp.float32),
        jax.ShapeDtypeStruct((NP, HH), jnp.float32),
        jax.ShapeDtypeStruct((NP, HID), jnp.float32),
        jax.ShapeDtypeStruct((NP, DE), jnp.float32),
    ]
    return pl.pallas_call(
        _proj_body,
        grid=(NP // TCB,),
        in_specs=[
            pl.BlockSpec((TCB, D), lambda i: (i, 0)),
            pl.BlockSpec((D, 4 * HID), lambda i: (0, 0)),
            pl.BlockSpec((1, 4 * HID), lambda i: (0, 0)),
            pl.BlockSpec((D, DE), lambda i: (0, 0)),
        ],
        out_specs=[
            pl.BlockSpec((TCB, HID), lambda i: (i, 0)),
            pl.BlockSpec((TCB, HID), lambda i: (i, 0)),
            pl.BlockSpec((TCB, HH), lambda i: (i, 0)),
            pl.BlockSpec((TCB, HH), lambda i: (i, 0)),
            pl.BlockSpec((TCB, HID), lambda i: (i, 0)),
            pl.BlockSpec((TCB, DE), lambda i: (i, 0)),
        ],
        out_shape=outs,
    )(h, wcat, bcat, we)


def _pass1_body(q_hbm, k_hbm, qw_hbm, ea_hbm, src2_hbm, dst2_hbm,
                s_out, agg_out,
                idx_src, idx_dst, qbufs, kbufs, qwbufs, eabufs,
                stage32, sbuf, agg_sp, semg, sems):
    c = lax.axis_index("c")
    s_ax = lax.axis_index("s")
    wid = c * 16 + s_ax
    ebase = wid * TE
    rbase = s_ax * RPT

    pltpu.sync_copy(src2_hbm.at[pl.ds(wid * NCH, NCH)], idx_src)
    pltpu.sync_copy(dst2_hbm.at[pl.ds(wid * NCH, NCH)], idx_dst)

    zf = jnp.zeros((16,), jnp.float32)

    def _zstage(r, _):
        stage32[r, pl.ds(0, 16)] = zf
        stage32[r, pl.ds(16, 16)] = zf
        return 0
    lax.fori_loop(0, CH, _zstage, 0)

    for z in range(RPT // CH):
        pltpu.sync_copy(stage32, agg_sp.at[pl.ds(rbase + z * CH, CH)])
    plsc.subcore_barrier()

    lanes = lax.iota(jnp.int32, 16)

    def _issue(ci, b):
        ci = lax.rem(ci, NCH)
        pltpu.async_copy(q_hbm.at[idx_dst.at[ci]], qbufs.at[b], semg.at[b])
        pltpu.async_copy(k_hbm.at[idx_src.at[ci]], kbufs.at[b], semg.at[b])
        pltpu.async_copy(qw_hbm.at[idx_dst.at[ci]], qwbufs.at[b], semg.at[b])
        pltpu.async_copy(ea_hbm.at[pl.ds(ebase + ci * CH, CH)],
                         eabufs.at[b], semg.at[b])

    def _drain(ci, b):
        ci = lax.rem(ci, NCH)
        pltpu.make_async_copy(q_hbm.at[idx_dst.at[ci]], qbufs.at[b],
                              semg.at[b]).wait()
        pltpu.make_async_copy(k_hbm.at[idx_src.at[ci]], kbufs.at[b],
                              semg.at[b]).wait()
        pltpu.make_async_copy(qw_hbm.at[idx_dst.at[ci]], qwbufs.at[b],
                              semg.at[b]).wait()
        pltpu.make_async_copy(ea_hbm.at[pl.ds(ebase + ci * CH, CH)],
                              eabufs.at[b], semg.at[b]).wait()

    def _slot(ci, b, first):
        qbuf, kbuf = qbufs.at[b], kbufs.at[b]
        qwbuf, eabuf = qwbufs.at[b], eabufs.at[b]
        _issue(ci + 1, 1 - b)
        _drain(ci, b)
        for g in range(CH // 16):
            ids = g * 16 + lanes
            gid = ebase + ci * CH + ids
            mask = gid < E

            def _dot(f, acc):
                col = jnp.full((16,), f, jnp.int32)
                qv = plsc.load_gather(qbuf, [ids, col])
                kv = plsc.load_gather(kbuf, [ids, col])
                return acc + qv * kv
            acc = lax.fori_loop(0, HID, _dot, jnp.zeros((16,), jnp.float32),
                                unroll=8)
            for f in range(DE):
                col = jnp.full((16,), f, jnp.int32)
                acc = acc + (plsc.load_gather(qwbuf, [ids, col])
                             * plsc.load_gather(eabuf, [ids, col]))
            sv = jnp.where(mask, jnp.exp(acc * SCALE), 0.0)
            sbuf[pl.ds(ci * CH + g * 16, 16)] = sv

        @pl.when(jnp.logical_not(first))
        def _():
            pltpu.make_async_copy(stage32, agg_sp.at[idx_dst.at[ci]],
                                  sems).wait()

        def _egroup(g2, _):
            wv = sbuf[pl.ds(ci * CH + g2 * 16, 16)]
            for j16 in range(16):
                j = g2 * 16 + j16
                w = wv[j16]
                stage32[j, pl.ds(0, 16)] = eabuf[j, :] * w
                stage32[j, pl.ds(16, 16)] = jnp.full((16,), w, jnp.float32)
            return 0
        lax.fori_loop(0, CH // 16, _egroup, 0)

        pltpu.async_copy(stage32, agg_sp.at[idx_dst.at[ci]], sems, add=True)

    _issue(0, 0)

    def _pair(ci2, _):
        _slot(2 * ci2, 0, ci2 == 0)
        _slot(2 * ci2 + 1, 1, False)
        return 0
    lax.fori_loop(0, NCH // 2, _pair, 0)

    _drain(NCH, 0)
    pltpu.make_async_copy(stage32, agg_sp.at[idx_dst.at[NCH - 1]],
                          sems).wait()
    pltpu.sync_copy(sbuf, s_out.at[pl.ds(ebase, TE)])

    plsc.subcore_barrier()
    pltpu.sync_copy(agg_sp.at[pl.ds(rbase, RPT)],
                    agg_out.at[c, pl.ds(rbase, RPT)])


def _pass1(q, k, qw, ea_p, src2, dst2):
    f = pl.kernel(
        _pass1_body,
        out_type=[
            jax.ShapeDtypeStruct((EP,), jnp.float32),
            jax.ShapeDtypeStruct((2, NP, 32), jnp.float32),
        ],
        mesh=_MESH,
        compiler_params=_SC_PARAMS,
        scratch_types=[
            pltpu.VMEM((NCH, CH), jnp.int32),
            pltpu.VMEM((NCH, CH), jnp.int32),
            pltpu.VMEM((2, CH, HID), jnp.float32),
            pltpu.VMEM((2, CH, HID), jnp.float32),
            pltpu.VMEM((2, CH, DE), jnp.float32),
            pltpu.VMEM((2, CH, DE), jnp.float32),
            pltpu.VMEM((CH, 32), jnp.float32),
            pltpu.VMEM((TE,), jnp.float32),
            pltpu.VMEM_SHARED((NP, 32), jnp.float32),
            pltpu.SemaphoreType.DMA((2,)),
            pltpu.SemaphoreType.DMA,
        ],
    )
    return f(q, k, qw, ea_p, src2, dst2)


def _pass2_body(vh_hbm, s_hbm, src2_hbm, dst2_hbm,
                acc_out,
                idx_src, idx_dst, vbufs, stage, sbuf, acc_sp, semv, sems):
    c = lax.axis_index("c")
    s_ax = lax.axis_index("s")
    wid = c * 16 + s_ax
    ebase = wid * TE
    rbase = s_ax * RPT

    pltpu.sync_copy(src2_hbm.at[pl.ds(wid * NCH, NCH)], idx_src)
    pltpu.sync_copy(dst2_hbm.at[pl.ds(wid * NCH, NCH)], idx_dst)
    pltpu.sync_copy(s_hbm.at[pl.ds(ebase, TE)], sbuf)

    zf = jnp.zeros((16,), jnp.float32)

    def _zstage(r, _):
        for b in range(HH // 16):
            stage[r, pl.ds(b * 16, 16)] = zf
        return 0
    lax.fori_loop(0, CH, _zstage, 0)
    for z in range(RPT // CH):
        pltpu.sync_copy(stage, acc_sp.at[pl.ds(rbase + z * CH, CH)])
    plsc.subcore_barrier()

    def _issue(ci, b):
        ci = lax.rem(ci, NCH)
        pltpu.async_copy(vh_hbm.at[idx_src.at[ci]], vbufs.at[b], semv.at[b])

    def _drain(ci, b):
        ci = lax.rem(ci, NCH)
        pltpu.make_async_copy(vh_hbm.at[idx_src.at[ci]], vbufs.at[b],
                              semv.at[b]).wait()

    def _slot(ci, b, first):
        vbuf = vbufs.at[b]
        _issue(ci + 1, 1 - b)
        _drain(ci, b)

        @pl.when(jnp.logical_not(first))
        def _():
            pltpu.make_async_copy(stage, acc_sp.at[idx_dst.at[ci]],
                                  sems).wait()

        def _egroup(g2, _):
            wv = sbuf[pl.ds(ci * CH + g2 * 16, 16)]
            for j16 in range(16):
                j = g2 * 16 + j16
                w = wv[j16]
                for b2 in range(HH // 16):
                    stage[j, pl.ds(b2 * 16, 16)] = \
                        vbuf[j, pl.ds(b2 * 16, 16)] * w
            return 0
        lax.fori_loop(0, CH // 16, _egroup, 0)
        pltpu.async_copy(stage, acc_sp.at[idx_dst.at[ci]], sems, add=True)

    _issue(0, 0)

    def _pair(ci2, _):
        _slot(2 * ci2, 0, ci2 == 0)
        _slot(2 * ci2 + 1, 1, False)
        return 0
    lax.fori_loop(0, NCH // 2, _pair, 0)

    _drain(NCH, 0)
    pltpu.make_async_copy(stage, acc_sp.at[idx_dst.at[NCH - 1]], sems).wait()
    plsc.subcore_barrier()
    pltpu.sync_copy(acc_sp.at[pl.ds(rbase, RPT)],
                    acc_out.at[c, pl.ds(rbase, RPT)])


def _pass2(vh, s, src2, dst2):
    f = pl.kernel(
        _pass2_body,
        out_type=jax.ShapeDtypeStruct((2, NP, HH), jnp.float32),
        mesh=_MESH,
        compiler_params=_SC_PARAMS,
        scratch_types=[
            pltpu.VMEM((NCH, CH), jnp.int32),
            pltpu.VMEM((NCH, CH), jnp.int32),
            pltpu.VMEM((2, CH, HH), jnp.float32),
            pltpu.VMEM((CH, HH), jnp.float32),
            pltpu.VMEM((TE,), jnp.float32),
            pltpu.VMEM_SHARED((NP, HH), jnp.float32),
            pltpu.SemaphoreType.DMA((2,)),
            pltpu.SemaphoreType.DMA,
        ],
    )
    return f(vh, s, src2, dst2)


def _combine_body(acclo_ref, acchi_ref, agg_ref, skip_ref,
                  wet_ref, out_ref):
    a = agg_ref[0] + agg_ref[1]
    den = a[:, 16:17]
    inv = 1.0 / (den + 1e-16)
    lo = (acclo_ref[0] + acclo_ref[1]) * inv
    hi = (acchi_ref[0] + acchi_ref[1]) * inv
    corr = jnp.dot(a[:, 0:DE], wet_ref[...],
                   preferred_element_type=jnp.float32)
    out_ref[...] = jax.nn.relu(
        jnp.concatenate([lo, hi], axis=1) + corr + skip_ref[...])


def _combine(acclo, acchi, agg, skip, wet):
    return pl.pallas_call(
        _combine_body,
        grid=(NP // TCB,),
        in_specs=[
            pl.BlockSpec((2, TCB, HH), lambda i: (0, i, 0)),
            pl.BlockSpec((2, TCB, HH), lambda i: (0, i, 0)),
            pl.BlockSpec((2, TCB, 32), lambda i: (0, i, 0)),
            pl.BlockSpec((TCB, HID), lambda i: (i, 0)),
            pl.BlockSpec((DE, HID), lambda i: (0, 0)),
        ],
        out_specs=pl.BlockSpec((TCB, HID), lambda i: (i, 0)),
        out_shape=jax.ShapeDtypeStruct((NP, HID), jnp.float32),
    )(acclo, acchi, agg, skip, wet)


def _readout_body(h_ref, w1_ref, b1_ref, w2_ref, b2_ref, out_ref):
    pooled = jnp.sum(h_ref[0:N, :], axis=0, keepdims=True) * (1.0 / N)
    r = jax.nn.relu(jnp.dot(pooled, w1_ref[...],
                            preferred_element_type=jnp.float32) + b1_ref[...])
    out_ref[...] = jnp.dot(r, w2_ref[...],
                           preferred_element_type=jnp.float32) + b2_ref[...]


def _readout(h, w1t, b1, w2t, b2):
    return pl.pallas_call(
        _readout_body,
        out_shape=jax.ShapeDtypeStruct((1, 1), jnp.float32),
    )(h, w1t, b1, w2t, b2)


def kernel(x, edge_index, edge_attr, layer_params, readout_params):
    src = edge_index[0]
    dst = edge_index[1]
    src2 = jnp.pad(src, (0, EP - E)).reshape(EP // CH, CH)
    dst2 = jnp.pad(dst, (0, EP - E)).reshape(EP // CH, CH)
    ea_p = jnp.pad(edge_attr, ((0, EP - E), (0, 0)))
    h = jnp.pad(x, ((0, NP - N), (0, 0)))

    for p in layer_params:
        wcat = jnp.concatenate(
            [p["Wq"].T, p["Wk"].T, p["Wv"].T, p["Ws"].T], axis=1)
        bcat = jnp.concatenate(
            [p["bq"], p["bk"], p["bv"], p["bs"]])[None, :]
        q, k, vlo, vhi, skip, qw = _proj(h, wcat, bcat, p["We"])
        s, agg = _pass1(q, k, qw, ea_p, src2, dst2)
        acclo = _pass2(vlo, s, src2, dst2)
        acchi = _pass2(vhi, s, src2, dst2)
        h = _combine(acclo, acchi, agg, skip, p["We"].T)

    rp = readout_params
    out = _readout(h, rp["W1"].T, rp["b1"][None, :], rp["W2"].T,
                   rp["b2"][None, :])
    return out[0]
